# baseline (speedup 1.0000x reference)
import functools

import jax
import jax.numpy as jnp
from jax import lax
from jax.experimental import pallas as pl
from jax.experimental.pallas import tpu as pltpu
from jax.experimental.pallas import tpu_sc as plsc

N = 10000
E = 320000
NC = 2
NS = 16
NW = NC * NS
EB = 128
E2 = 327680
NB = E2 // EB // NW
NBH = NB // 2
NPAD = NS * 640
RCH = 40
DGW = 128

_mesh = plsc.VectorSubcoreMesh(core_axis_name="c", subcore_axis_name="s")


@functools.partial(
    pl.kernel,
    mesh=_mesh,
    out_type=jax.ShapeDtypeStruct((NC, NPAD, DGW), jnp.float32),
    scratch_types=[
        pltpu.VMEM((NB, EB), jnp.int32),
        pltpu.VMEM((EB, DGW), jnp.float32),
        pltpu.VMEM((RCH, DGW), jnp.float32),
        pltpu.VMEM_SHARED((NPAD, DGW), jnp.float32),
    ],
)
def _deg_sc(dst_hbm, ones_hbm, zeros_hbm, degp_hbm, didx_v, ones_v, chunk_v,
            acc_sp):
    cid = lax.axis_index("c")
    sid = lax.axis_index("s")
    wid = sid * NC + cid

    pltpu.sync_copy(ones_hbm, ones_v)
    pltpu.sync_copy(zeros_hbm, chunk_v)

    def zchunk(k, _):
        pltpu.sync_copy(chunk_v, acc_sp.at[pl.ds(sid * 640 + k * RCH, RCH)])
        return 0

    lax.fori_loop(0, 640 // RCH, zchunk, 0)
    plsc.subcore_barrier()

    pltpu.sync_copy(dst_hbm.at[pl.ds(wid * NB, NB)], didx_v)

    def body(j, _):
        pltpu.sync_copy(ones_v, acc_sp.at[didx_v.at[j]], add=True)
        return 0

    lax.fori_loop(0, NB, body, 0)
    plsc.subcore_barrier()

    def wout(k, _):
        r0 = sid * 640 + k * RCH
        pltpu.sync_copy(acc_sp.at[pl.ds(r0, RCH)], chunk_v)
        pltpu.sync_copy(chunk_v, degp_hbm.at[cid, pl.ds(r0, RCH)])
        return 0

    lax.fori_loop(0, 640 // RCH, wout, 0)


U = 2
ROUNDS_H = NBH // U


def _make_prop(d):
    @functools.partial(
        pl.kernel,
        mesh=_mesh,
        out_type=jax.ShapeDtypeStruct((NC, NPAD, d), jnp.float32),
        scratch_types=[
            pltpu.VMEM((NBH, EB), jnp.int32),
            pltpu.VMEM((NBH, EB), jnp.int32),
            pltpu.VMEM((EB, d), jnp.float32),
            pltpu.VMEM((EB, d), jnp.float32),
            pltpu.VMEM((RCH, d), jnp.float32),
            pltpu.SemaphoreType.DMA,
            pltpu.SemaphoreType.DMA,
            pltpu.SemaphoreType.DMA,
            pltpu.SemaphoreType.DMA,
            pltpu.VMEM_SHARED((NPAD, d), jnp.float32),
        ],
    )
    def _prop_sc(src_hbm, dst_hbm, y_hbm, zeros_hbm, zp_hbm,
                 sidx_v, didx_v, rows0, rows1, chunk_v,
                 gsem0, gsem1, ssem0, ssem1, acc_sp):
        rows = (rows0, rows1)
        gsem = (gsem0, gsem1)
        ssem = (ssem0, ssem1)
        cid = lax.axis_index("c")
        sid = lax.axis_index("s")
        wid = sid * NC + cid

        pltpu.sync_copy(zeros_hbm, chunk_v)

        def zchunk(k, _):
            pltpu.sync_copy(chunk_v,
                            acc_sp.at[pl.ds(sid * 640 + k * RCH, RCH)])
            return 0

        lax.fori_loop(0, 640 // RCH, zchunk, 0)
        plsc.subcore_barrier()

        for half in range(2):
            base = wid * NB + half * NBH
            pltpu.sync_copy(src_hbm.at[pl.ds(base, NBH)], sidx_v)
            pltpu.sync_copy(dst_hbm.at[pl.ds(base, NBH)], didx_v)

            def round_body(j, _):
                pltpu.async_copy(
                    y_hbm.at[sidx_v.at[j]], rows[0], gsem[0]).wait()
                pltpu.sync_copy(rows[0], acc_sp.at[didx_v.at[j]], add=True)
                return 0

            lax.fori_loop(0, NBH, round_body, 0)
        plsc.subcore_barrier()

        def wout(k, _):
            r0 = sid * 640 + k * RCH
            pltpu.sync_copy(acc_sp.at[pl.ds(r0, RCH)], chunk_v)
            pltpu.sync_copy(chunk_v, zp_hbm.at[cid, pl.ds(r0, RCH)])
            return 0

        lax.fori_loop(0, 640 // RCH, wout, 0)

    return _prop_sc


_prop128 = _make_prop(128)


BR = 1000
GRID = N // BR


def _dis_from(degp_ref):
    deg = degp_ref[0, :, 0] + degp_ref[1, :, 0] + 1.0
    return lax.rsqrt(deg)[:, None]


def _tc1_body(degp_ref, x_ref, w_ref, y_ref):
    dis = _dis_from(degp_ref)
    xw = jnp.dot(x_ref[...], w_ref[...], preferred_element_type=jnp.float32)
    y_ref[...] = xw * dis


def _tc_mid_body(degp_ref, zp_ref, y_ref, b_ref, w_ref, o_ref):
    dis = _dis_from(degp_ref)
    h = jnp.maximum(dis * (zp_ref[0] + zp_ref[1] + y_ref[...]) + b_ref[...], 0.0)
    o_ref[...] = jnp.dot(h, w_ref[...], preferred_element_type=jnp.float32) * dis


def _tc4_body(degp_ref, zp_ref, y_ref, b_ref, o_ref):
    dis = _dis_from(degp_ref)
    o = dis * (zp_ref[0] + zp_ref[1] + y_ref[...]) + b_ref[...]
    col = lax.broadcasted_iota(jnp.int32, o.shape, 1)
    valid = col < 40
    om = jnp.where(valid, o, -1e30)
    m = jnp.max(om, axis=1, keepdims=True)
    e = jnp.where(valid, jnp.exp(o - m), 0.0)
    lse = jnp.log(jnp.sum(e, axis=1, keepdims=True))
    o_ref[...] = (o - m - lse)[:, :40]


def _degp_spec():
    return pl.BlockSpec((2, BR, DGW), lambda i: (0, i, 0))


def _tc1(degp, x, w):
    d = x.shape[1]
    dn = w.shape[1]
    return pl.pallas_call(
        _tc1_body,
        grid=(GRID,),
        in_specs=[
            _degp_spec(),
            pl.BlockSpec((BR, d), lambda i: (i, 0)),
            pl.BlockSpec((d, dn), lambda i: (0, 0)),
        ],
        out_specs=pl.BlockSpec((BR, dn), lambda i: (i, 0)),
        out_shape=jax.ShapeDtypeStruct((N, dn), jnp.float32),
    )(degp, x, w)


def _tc_mid(degp, zp, y, b, w):
    d = y.shape[1]
    dn = w.shape[1]
    return pl.pallas_call(
        _tc_mid_body,
        grid=(GRID,),
        in_specs=[
            _degp_spec(),
            pl.BlockSpec((2, BR, d), lambda i: (0, i, 0)),
            pl.BlockSpec((BR, d), lambda i: (i, 0)),
            pl.BlockSpec((1, d), lambda i: (0, 0)),
            pl.BlockSpec((d, dn), lambda i: (0, 0)),
        ],
        out_specs=pl.BlockSpec((BR, dn), lambda i: (i, 0)),
        out_shape=jax.ShapeDtypeStruct((N, dn), jnp.float32),
    )(degp, zp, y, b, w)


def _tc4(degp, zp, y, b):
    d = y.shape[1]
    return pl.pallas_call(
        _tc4_body,
        grid=(GRID,),
        in_specs=[
            _degp_spec(),
            pl.BlockSpec((2, BR, d), lambda i: (0, i, 0)),
            pl.BlockSpec((BR, d), lambda i: (i, 0)),
            pl.BlockSpec((1, d), lambda i: (0, 0)),
        ],
        out_specs=pl.BlockSpec((BR, 40), lambda i: (i, 0)),
        out_shape=jax.ShapeDtypeStruct((N, 40), jnp.float32),
    )(degp, zp, y, b)


def _padn(y):
    return jnp.pad(y, ((0, NPAD - N), (0, 0)))


def kernel(x, edge_index, W1, b1, W2, b2, W3, b3):
    fake = jnp.arange(E2 - E, dtype=jnp.int32) % (NPAD - N) + N
    eix = jnp.concatenate([edge_index, jnp.stack([fake, fake])], axis=1)
    src2d = eix[0].reshape(E2 // EB, EB)
    dst2d = eix[1].reshape(E2 // EB, EB)
    ones_c = jnp.ones((EB, DGW), jnp.float32)
    zeros_c = jnp.zeros((RCH, 128), jnp.float32)

    degp = _deg_sc(dst2d, ones_c, zeros_c)

    y1 = _tc1(degp, x, W1)
    zp1 = _prop128(src2d, dst2d, _padn(y1), zeros_c)

    y2 = _tc_mid(degp, zp1, y1, b1.reshape(1, -1), W2)
    zp2 = _prop128(src2d, dst2d, _padn(y2), zeros_c)

    W3p = jnp.pad(W3, ((0, 0), (0, 128 - W3.shape[1])))
    b3p = jnp.pad(b3, (0, 128 - b3.shape[0]))
    y3 = _tc_mid(degp, zp2, y2, b2.reshape(1, -1), W3p)
    zp3 = _prop128(src2d, dst2d, _padn(y3), zeros_c)

    return _tc4(degp, zp3, y3, b3p.reshape(1, -1))


# speedup vs baseline: 1.2325x; 1.2325x over previous
import functools

import jax
import jax.numpy as jnp
from jax import lax
from jax.experimental import pallas as pl
from jax.experimental.pallas import tpu as pltpu
from jax.experimental.pallas import tpu_sc as plsc

N = 10000
E = 320000
NC = 2
NS = 16
NW = NC * NS
EB = 128
E2 = 327680
NB = E2 // EB // NW
NBH = NB // 2
NPAD = NS * 640
RCH = 40
DGW = 128

_mesh = plsc.VectorSubcoreMesh(core_axis_name="c", subcore_axis_name="s")


@functools.partial(
    pl.kernel,
    mesh=_mesh,
    out_type=jax.ShapeDtypeStruct((NC, NPAD, DGW), jnp.float32),
    scratch_types=[
        pltpu.VMEM((NB, EB), jnp.int32),
        pltpu.VMEM((EB, DGW), jnp.float32),
        pltpu.VMEM((RCH, DGW), jnp.float32),
        pltpu.VMEM_SHARED((NPAD, DGW), jnp.float32),
    ],
)
def _deg_sc(dst_hbm, ones_hbm, zeros_hbm, degp_hbm, didx_v, ones_v, chunk_v,
            acc_sp):
    cid = lax.axis_index("c")
    sid = lax.axis_index("s")
    wid = sid * NC + cid

    pltpu.sync_copy(ones_hbm, ones_v)
    pltpu.sync_copy(zeros_hbm, chunk_v)

    def zchunk(k, _):
        pltpu.sync_copy(chunk_v, acc_sp.at[pl.ds(sid * 640 + k * RCH, RCH)])
        return 0

    lax.fori_loop(0, 640 // RCH, zchunk, 0)
    plsc.subcore_barrier()

    pltpu.sync_copy(dst_hbm.at[pl.ds(wid * NB, NB)], didx_v)

    def body(j, _):
        pltpu.sync_copy(ones_v, acc_sp.at[didx_v.at[j]], add=True)
        return 0

    lax.fori_loop(0, NB, body, 0)
    plsc.subcore_barrier()

    def wout(k, _):
        r0 = sid * 640 + k * RCH
        pltpu.sync_copy(acc_sp.at[pl.ds(r0, RCH)], chunk_v)
        pltpu.sync_copy(chunk_v, degp_hbm.at[cid, pl.ds(r0, RCH)])
        return 0

    lax.fori_loop(0, 640 // RCH, wout, 0)


U = 2
ROUNDS_H = NBH // U


def _make_prop(d):
    @functools.partial(
        pl.kernel,
        mesh=_mesh,
        out_type=jax.ShapeDtypeStruct((NC, NPAD, d), jnp.float32),
        scratch_types=[
            pltpu.VMEM((NBH, EB), jnp.int32),
            pltpu.VMEM((NBH, EB), jnp.int32),
            pltpu.VMEM((EB, d), jnp.float32),
            pltpu.VMEM((EB, d), jnp.float32),
            pltpu.VMEM((RCH, d), jnp.float32),
            pltpu.SemaphoreType.DMA,
            pltpu.SemaphoreType.DMA,
            pltpu.SemaphoreType.DMA,
            pltpu.SemaphoreType.DMA,
            pltpu.VMEM_SHARED((NPAD, d), jnp.float32),
        ],
    )
    def _prop_sc(src_hbm, dst_hbm, y_hbm, zeros_hbm, zp_hbm,
                 sidx_v, didx_v, rows0, rows1, chunk_v,
                 gsem0, gsem1, ssem0, ssem1, acc_sp):
        rows = (rows0, rows1)
        gsem = (gsem0, gsem1)
        ssem = (ssem0, ssem1)
        cid = lax.axis_index("c")
        sid = lax.axis_index("s")
        wid = sid * NC + cid

        pltpu.sync_copy(zeros_hbm, chunk_v)

        def zchunk(k, _):
            pltpu.sync_copy(chunk_v,
                            acc_sp.at[pl.ds(sid * 640 + k * RCH, RCH)])
            return 0

        lax.fori_loop(0, 640 // RCH, zchunk, 0)
        plsc.subcore_barrier()

        for half in range(2):
            base = wid * NB + half * NBH
            pltpu.sync_copy(src_hbm.at[pl.ds(base, NBH)], sidx_v)
            pltpu.sync_copy(dst_hbm.at[pl.ds(base, NBH)], didx_v)

            pltpu.async_copy(y_hbm.at[sidx_v.at[0]], rows[0], gsem[0])

            def round_body(k, _):
                j0 = 2 * k
                j1 = 2 * k + 1
                jn = jnp.minimum(j0 + 2, NBH - 1)
                pltpu.make_async_copy(
                    y_hbm.at[sidx_v.at[j0]], rows[0], gsem[0]).wait()
                pltpu.async_copy(y_hbm.at[sidx_v.at[j1]], rows[1], gsem[1])
                pltpu.sync_copy(rows[0], acc_sp.at[didx_v.at[j0]], add=True)
                pltpu.make_async_copy(
                    y_hbm.at[sidx_v.at[j1]], rows[1], gsem[1]).wait()
                pltpu.async_copy(y_hbm.at[sidx_v.at[jn]], rows[0], gsem[0])
                pltpu.sync_copy(rows[1], acc_sp.at[didx_v.at[j1]], add=True)
                return 0

            lax.fori_loop(0, ROUNDS_H, round_body, 0)
            pltpu.make_async_copy(
                y_hbm.at[sidx_v.at[NBH - 1]], rows[0], gsem[0]).wait()
        plsc.subcore_barrier()

        def wout(k, _):
            r0 = sid * 640 + k * RCH
            pltpu.sync_copy(acc_sp.at[pl.ds(r0, RCH)], chunk_v)
            pltpu.sync_copy(chunk_v, zp_hbm.at[cid, pl.ds(r0, RCH)])
            return 0

        lax.fori_loop(0, 640 // RCH, wout, 0)

    return _prop_sc


_prop128 = _make_prop(128)


BR = 1000
GRID = N // BR


def _dis_from(degp_ref):
    deg = degp_ref[0, :, 0] + degp_ref[1, :, 0] + 1.0
    return lax.rsqrt(deg)[:, None]


def _tc1_body(degp_ref, x_ref, w_ref, y_ref):
    dis = _dis_from(degp_ref)
    xw = jnp.dot(x_ref[...], w_ref[...], preferred_element_type=jnp.float32)
    y_ref[...] = xw * dis


def _tc_mid_body(degp_ref, zp_ref, y_ref, b_ref, w_ref, o_ref):
    dis = _dis_from(degp_ref)
    h = jnp.maximum(dis * (zp_ref[0] + zp_ref[1] + y_ref[...]) + b_ref[...], 0.0)
    o_ref[...] = jnp.dot(h, w_ref[...], preferred_element_type=jnp.float32) * dis


def _tc4_body(degp_ref, zp_ref, y_ref, b_ref, o_ref):
    dis = _dis_from(degp_ref)
    o = dis * (zp_ref[0] + zp_ref[1] + y_ref[...]) + b_ref[...]
    col = lax.broadcasted_iota(jnp.int32, o.shape, 1)
    valid = col < 40
    om = jnp.where(valid, o, -1e30)
    m = jnp.max(om, axis=1, keepdims=True)
    e = jnp.where(valid, jnp.exp(o - m), 0.0)
    lse = jnp.log(jnp.sum(e, axis=1, keepdims=True))
    o_ref[...] = (o - m - lse)[:, :40]


def _degp_spec():
    return pl.BlockSpec((2, BR, DGW), lambda i: (0, i, 0))


def _tc1(degp, x, w):
    d = x.shape[1]
    dn = w.shape[1]
    return pl.pallas_call(
        _tc1_body,
        grid=(GRID,),
        in_specs=[
            _degp_spec(),
            pl.BlockSpec((BR, d), lambda i: (i, 0)),
            pl.BlockSpec((d, dn), lambda i: (0, 0)),
        ],
        out_specs=pl.BlockSpec((BR, dn), lambda i: (i, 0)),
        out_shape=jax.ShapeDtypeStruct((N, dn), jnp.float32),
    )(degp, x, w)


def _tc_mid(degp, zp, y, b, w):
    d = y.shape[1]
    dn = w.shape[1]
    return pl.pallas_call(
        _tc_mid_body,
        grid=(GRID,),
        in_specs=[
            _degp_spec(),
            pl.BlockSpec((2, BR, d), lambda i: (0, i, 0)),
            pl.BlockSpec((BR, d), lambda i: (i, 0)),
            pl.BlockSpec((1, d), lambda i: (0, 0)),
            pl.BlockSpec((d, dn), lambda i: (0, 0)),
        ],
        out_specs=pl.BlockSpec((BR, dn), lambda i: (i, 0)),
        out_shape=jax.ShapeDtypeStruct((N, dn), jnp.float32),
    )(degp, zp, y, b, w)


def _tc4(degp, zp, y, b):
    d = y.shape[1]
    return pl.pallas_call(
        _tc4_body,
        grid=(GRID,),
        in_specs=[
            _degp_spec(),
            pl.BlockSpec((2, BR, d), lambda i: (0, i, 0)),
            pl.BlockSpec((BR, d), lambda i: (i, 0)),
            pl.BlockSpec((1, d), lambda i: (0, 0)),
        ],
        out_specs=pl.BlockSpec((BR, 40), lambda i: (i, 0)),
        out_shape=jax.ShapeDtypeStruct((N, 40), jnp.float32),
    )(degp, zp, y, b)


def _padn(y):
    return jnp.pad(y, ((0, NPAD - N), (0, 0)))


def kernel(x, edge_index, W1, b1, W2, b2, W3, b3):
    fake = jnp.arange(E2 - E, dtype=jnp.int32) % (NPAD - N) + N
    eix = jnp.concatenate([edge_index, jnp.stack([fake, fake])], axis=1)
    src2d = eix[0].reshape(E2 // EB, EB)
    dst2d = eix[1].reshape(E2 // EB, EB)
    ones_c = jnp.ones((EB, DGW), jnp.float32)
    zeros_c = jnp.zeros((RCH, 128), jnp.float32)

    degp = _deg_sc(dst2d, ones_c, zeros_c)

    y1 = _tc1(degp, x, W1)
    zp1 = _prop128(src2d, dst2d, _padn(y1), zeros_c)

    y2 = _tc_mid(degp, zp1, y1, b1.reshape(1, -1), W2)
    zp2 = _prop128(src2d, dst2d, _padn(y2), zeros_c)

    W3p = jnp.pad(W3, ((0, 0), (0, 128 - W3.shape[1])))
    b3p = jnp.pad(b3, (0, 128 - b3.shape[0]))
    y3 = _tc_mid(degp, zp2, y2, b2.reshape(1, -1), W3p)
    zp3 = _prop128(src2d, dst2d, _padn(y3), zeros_c)

    return _tc4(degp, zp3, y3, b3p.reshape(1, -1))
